# baseline (speedup 1.0000x reference)
import functools
import math

import jax
import jax.numpy as jnp
from jax import lax
from jax.experimental import pallas as pl
from jax.experimental.pallas import tpu as pltpu
from jax.experimental.pallas import tpu_sc as plsc

D_MODEL = 64
SEQ_LEN = 2048
BATCH = 4
NC = 2
NS = 16
NW = NC * NS
ROWS_PER_W = (SEQ_LEN * BATCH) // NW
CHUNK = 128
SEQ_PER_W = ROWS_PER_W // BATCH
SEQ_PER_CHUNK = CHUNK // BATCH
LANES = 16
VPD = D_MODEL // LANES
SCALE = math.sqrt(D_MODEL)


def _fma_chunk(buf, out_b, par_s, par_off, pe_v, pe_off):
    def body(s, carry):
        pvals = [pe_v[pe_off + s, pl.ds(j * LANES, LANES)] for j in range(VPD)]
        for b in range(BATCH):
            r = s * BATCH + b
            off = par_s[pl.ds(par_off + r, LANES)][0]
            for j in range(VPD):
                out_b[r, pl.ds(j * LANES, LANES)] = (
                    buf[r, pl.ds(off + j * LANES, LANES)] * SCALE + pvals[j]
                )
        return carry

    lax.fori_loop(0, SEQ_PER_CHUNK, body, 0, unroll=False)


def _emb_body(w_hbm, xp_hbm, par_hbm, pe_hbm, out_hbm,
              idx_v, par_s, buf0, buf1, out0, out1, pe_v,
              sem0, sem1, sem2):
    wid = lax.axis_index("s") * NC + lax.axis_index("c")
    base = wid * ROWS_PER_W

    pltpu.sync_copy(xp_hbm.at[pl.ds(base, ROWS_PER_W)], idx_v)
    g0 = pltpu.async_copy(w_hbm.at[idx_v.at[pl.ds(0, CHUNK)]], buf0, sem0)
    g1 = pltpu.async_copy(w_hbm.at[idx_v.at[pl.ds(CHUNK, CHUNK)]], buf1, sem1)
    pltpu.sync_copy(par_hbm.at[pl.ds(base, ROWS_PER_W)],
                    par_s.at[pl.ds(0, ROWS_PER_W)])
    pltpu.sync_copy(pe_hbm.at[pl.ds(wid * SEQ_PER_W, SEQ_PER_W)], pe_v)

    g0.wait()
    _fma_chunk(buf0, out0, par_s, 0, pe_v, 0)
    st0 = pltpu.async_copy(out0, out_hbm.at[pl.ds(base, CHUNK)], sem2)
    g1.wait()
    _fma_chunk(buf1, out1, par_s, CHUNK, pe_v, SEQ_PER_CHUNK)
    pltpu.sync_copy(out1, out_hbm.at[pl.ds(base + CHUNK, CHUNK)])
    st0.wait()


_emb_lookup = functools.partial(
    pl.kernel,
    out_type=jax.ShapeDtypeStruct((SEQ_LEN * BATCH, D_MODEL), jnp.float32),
    mesh=plsc.VectorSubcoreMesh(core_axis_name="c", subcore_axis_name="s"),
    scratch_types=[
        pltpu.VMEM((ROWS_PER_W,), jnp.int32),
        pltpu.VMEM((ROWS_PER_W + LANES,), jnp.int32),
        pltpu.VMEM((CHUNK, 2 * D_MODEL), jnp.float32),
        pltpu.VMEM((CHUNK, 2 * D_MODEL), jnp.float32),
        pltpu.VMEM((CHUNK, D_MODEL), jnp.float32),
        pltpu.VMEM((CHUNK, D_MODEL), jnp.float32),
        pltpu.VMEM((SEQ_PER_W, D_MODEL), jnp.float32),
        pltpu.SemaphoreType.DMA,
        pltpu.SemaphoreType.DMA,
        pltpu.SemaphoreType.DMA,
    ],
)(_emb_body)


@jax.jit
def kernel(x, weight, pe):
    s, b = x.shape
    d = weight.shape[1]
    w128 = weight.reshape(-1, 2 * d)
    x1d = x.reshape(-1).astype(jnp.int32)
    xpair = x1d >> 1
    xpar = (x1d & 1) << 6
    pe2d = pe[:s, 0, :]
    out = _emb_lookup(w128, xpair, xpar, pe2d)
    return out.reshape(s, b, d)


# speedup vs baseline: 1.1322x; 1.1322x over previous
import functools
import math

import jax
import jax.numpy as jnp
from jax import lax
from jax.experimental import pallas as pl
from jax.experimental.pallas import tpu as pltpu
from jax.experimental.pallas import tpu_sc as plsc

D_MODEL = 64
SEQ_LEN = 2048
BATCH = 4
NC = 2
NS = 16
NW = NC * NS
ROWS_PER_W = (SEQ_LEN * BATCH) // NW
CHUNK = 128
SEQ_PER_W = ROWS_PER_W // BATCH
SEQ_PER_CHUNK = CHUNK // BATCH
LANES = 16
VPD = D_MODEL // LANES
SCALE = math.sqrt(D_MODEL)


def _fma_chunk(buf, out_b, pe_v, pe_off):
    def body(s, carry):
        pvals = [pe_v[pe_off + s, pl.ds(j * LANES, LANES)] for j in range(VPD)]
        for b in range(BATCH):
            r = s * BATCH + b
            for j in range(VPD):
                sl = pl.ds(j * LANES, LANES)
                out_b[r, sl] = buf[r, sl] * SCALE + pvals[j]
        return carry

    lax.fori_loop(0, SEQ_PER_CHUNK, body, 0, unroll=False)


def _emb_body(w_hbm, x_hbm, pe_hbm, out_hbm,
              idx_v, buf0, buf1, out0, out1, pe_v, sem0, sem1, sem2):
    wid = lax.axis_index("s") * NC + lax.axis_index("c")
    base = wid * ROWS_PER_W

    pltpu.sync_copy(x_hbm.at[pl.ds(base, ROWS_PER_W)], idx_v)
    g0 = pltpu.async_copy(w_hbm.at[idx_v.at[pl.ds(0, CHUNK)]], buf0, sem0)
    g1 = pltpu.async_copy(w_hbm.at[idx_v.at[pl.ds(CHUNK, CHUNK)]], buf1, sem1)
    pltpu.sync_copy(pe_hbm.at[pl.ds(wid * SEQ_PER_W, SEQ_PER_W)], pe_v)

    g0.wait()
    _fma_chunk(buf0, out0, pe_v, 0)
    st0 = pltpu.async_copy(out0, out_hbm.at[pl.ds(base, CHUNK)], sem2)
    g1.wait()
    _fma_chunk(buf1, out1, pe_v, SEQ_PER_CHUNK)
    pltpu.sync_copy(out1, out_hbm.at[pl.ds(base + CHUNK, CHUNK)])
    st0.wait()


_emb_lookup = functools.partial(
    pl.kernel,
    out_type=jax.ShapeDtypeStruct((SEQ_LEN * BATCH, D_MODEL), jnp.float32),
    mesh=plsc.VectorSubcoreMesh(core_axis_name="c", subcore_axis_name="s"),
    scratch_types=[
        pltpu.VMEM((ROWS_PER_W,), jnp.int32),
        pltpu.VMEM((CHUNK, 2 * D_MODEL), jnp.float32),
        pltpu.VMEM((CHUNK, 2 * D_MODEL), jnp.float32),
        pltpu.VMEM((CHUNK, D_MODEL), jnp.float32),
        pltpu.VMEM((CHUNK, D_MODEL), jnp.float32),
        pltpu.VMEM((SEQ_PER_W, D_MODEL), jnp.float32),
        pltpu.SemaphoreType.DMA,
        pltpu.SemaphoreType.DMA,
        pltpu.SemaphoreType.DMA,
    ],
)(_emb_body)


@jax.jit
def kernel(x, weight, pe):
    s, b = x.shape
    d = weight.shape[1]
    wpad = lax.pad(weight, jnp.float32(0), ((0, 0, 0), (0, d, 0)))
    x1d = x.reshape(-1).astype(jnp.int32)
    pe2d = pe[:s, 0, :]
    out = _emb_lookup(wpad, x1d, pe2d)
    return out.reshape(s, b, d)
